# baseline (speedup 1.0000x reference)
import functools

import jax
import jax.numpy as jnp
from jax import lax
from jax.experimental import pallas as pl
from jax.experimental.pallas import tpu as pltpu
from jax.experimental.pallas import tpu_sc as plsc

N_NODES = 10000
N_FEATS = 256
HALF = 128
NC, NS = 2, 16
ROWS_PER_TILE = N_NODES // NS
K = 64
NCHUNK = 160
BLK = 16
NBLK = NCHUNK // BLK
E_PAD = NS * NCHUNK * K
NNZF = 256000
NNZF_PER_TILE = NNZF // NS


def _densify_call(frows, fcols, fvals):
    mesh = plsc.VectorSubcoreMesh(core_axis_name="c", subcore_axis_name="s")

    @functools.partial(
        pl.kernel,
        out_type=jax.ShapeDtypeStruct((NC, N_FEATS, 1, HALF), jnp.float32),
        mesh=mesh,
        compiler_params=pltpu.CompilerParams(needs_layout_passes=False),
        scratch_types=[
            pltpu.VMEM((NNZF_PER_TILE,), jnp.int32),
            pltpu.VMEM((NNZF_PER_TILE,), jnp.int32),
            pltpu.VMEM((NNZF_PER_TILE,), jnp.float32),
            pltpu.VMEM((N_FEATS, 1, HALF), jnp.float32),
            pltpu.VMEM((HALF,), jnp.int32),
            pltpu.VMEM((HALF,), jnp.int32),
            pltpu.VMEM_SHARED((N_FEATS, 1, HALF), jnp.float32),
        ],
    )
    def k(frows_hbm, fcols_hbm, fvals_hbm, out_hbm,
          rbuf, cbuf, vbuf, acc, idx_lo, idx_hi, shacc):
        c = lax.axis_index("c")
        s = lax.axis_index("s")
        for f in range(HALF // 16):
            idx_lo[pl.ds(f * 16, 16)] = lax.iota(jnp.int32, 16) + f * 16
            idx_hi[pl.ds(f * 16, 16)] = (
                lax.iota(jnp.int32, 16) + (HALF + f * 16))

        def zrow(i, _):
            for f in range(HALF // 16):
                acc[i, 0, pl.ds(f * 16, 16)] = jnp.zeros((16,), jnp.float32)
            return 0
        lax.fori_loop(0, N_FEATS, zrow, 0)

        @pl.when(s == 0)
        def _():
            pltpu.sync_copy(acc, shacc)

        pltpu.sync_copy(frows_hbm.at[pl.ds(s * NNZF_PER_TILE,
                                           NNZF_PER_TILE)], rbuf)
        pltpu.sync_copy(fcols_hbm.at[pl.ds(s * NNZF_PER_TILE,
                                           NNZF_PER_TILE)], cbuf)
        pltpu.sync_copy(fvals_hbm.at[pl.ds(s * NNZF_PER_TILE,
                                           NNZF_PER_TILE)], vbuf)

        def step(i, _):
            off = i * 16
            rv = rbuf[pl.ds(off, 16)]
            cv = cbuf[pl.ds(off, 16)]
            vv = vbuf[pl.ds(off, 16)]
            lc = cv - c * HALF
            m = (lc >= 0) & (lc < HALF)
            lcc = jnp.minimum(jnp.maximum(lc, 0), HALF - 1)
            plsc.addupdate_scatter(acc, [rv, jnp.zeros((16,), jnp.int32),
                                         lcc], vv, mask=m)
            return 0
        lax.fori_loop(0, NNZF_PER_TILE // 16, step, 0)

        plsc.subcore_barrier()
        pltpu.sync_copy(acc.at[pl.ds(0, HALF)], shacc.at[idx_lo], add=True)
        pltpu.sync_copy(acc.at[pl.ds(HALF, HALF)], shacc.at[idx_hi], add=True)
        plsc.subcore_barrier()

        @pl.when(s == 0)
        def _():
            pltpu.sync_copy(shacc, out_hbm.at[c])

    return k(frows, fcols, fvals)


def _spmm_call(src, rows3, cols3, vals3, src_rows):
    mesh = plsc.VectorSubcoreMesh(core_axis_name="c", subcore_axis_name="s")
    clip = src_rows < N_NODES

    @functools.partial(
        pl.kernel,
        out_type=jax.ShapeDtypeStruct((NC, N_NODES, 1, HALF), jnp.float32),
        mesh=mesh,
        compiler_params=pltpu.CompilerParams(use_tc_tiling_on_sc=False,
                                             needs_layout_passes=False),
        scratch_types=[
            pltpu.VMEM((BLK, K), jnp.int32),
            pltpu.VMEM((BLK, K), jnp.int32),
            pltpu.VMEM((BLK, K), jnp.float32),
            pltpu.VMEM((K, 1, HALF), jnp.float32),
            pltpu.VMEM((125, 1, HALF), jnp.float32),
            pltpu.VMEM_SHARED((N_NODES, 1, HALF), jnp.float32),
        ],
    )
    def k(src_hbm, rows_hbm, cols_hbm, vals_hbm, out_hbm,
          rbuf, cbuf, vbuf, gbuf, zbuf, acc):
        c = lax.axis_index("c")
        s = lax.axis_index("s")

        def zrow(i, _):
            for f in range(HALF // 16):
                zbuf[i, 0, pl.ds(f * 16, 16)] = jnp.zeros((16,), jnp.float32)
            return 0
        lax.fori_loop(0, 125, zrow, 0)
        for j in range(5):
            pltpu.sync_copy(
                zbuf, acc.at[pl.ds(s * ROWS_PER_TILE + j * 125, 125)])
        plsc.subcore_barrier()

        def block(b, _):
            pltpu.sync_copy(rows_hbm.at[s, b], rbuf)
            pltpu.sync_copy(cols_hbm.at[s, b], cbuf)
            pltpu.sync_copy(vals_hbm.at[s, b], vbuf)

            if clip:
                def fixup(i, _):
                    for j in range(K // 16):
                        cseg = cbuf[i, pl.ds(j * 16, 16)]
                        vseg = vbuf[i, pl.ds(j * 16, 16)]
                        m = cseg < src_rows
                        cbuf[i, pl.ds(j * 16, 16)] = jnp.minimum(
                            cseg, jnp.full((16,), src_rows - 1, jnp.int32))
                        vbuf[i, pl.ds(j * 16, 16)] = jnp.where(
                            m, vseg, jnp.zeros((16,), jnp.float32))
                    return 0
                lax.fori_loop(0, BLK, fixup, 0)

            def chunk(i, _):
                pltpu.sync_copy(src_hbm.at[c].at[cbuf.at[i]], gbuf)

                def group(j, _):
                    vseg = vbuf[i, pl.ds(j * 16, 16)]
                    for l in range(16):
                        sval = vseg[l]
                        e = j * 16 + l
                        for f in range(HALF // 16):
                            gbuf[e, 0, pl.ds(f * 16, 16)] = (
                                gbuf[e, 0, pl.ds(f * 16, 16)] * sval)
                    return 0
                lax.fori_loop(0, K // 16, group, 0)

                pltpu.sync_copy(gbuf, acc.at[rbuf.at[i]], add=True)
                return 0
            lax.fori_loop(0, BLK, chunk, 0)
            return 0
        lax.fori_loop(0, NBLK, block, 0)

        plsc.subcore_barrier()
        pltpu.sync_copy(acc.at[pl.ds(s * ROWS_PER_TILE, ROWS_PER_TILE)],
                        out_hbm.at[c, pl.ds(s * ROWS_PER_TILE, ROWS_PER_TILE)])

    return k(src, rows3, cols3, vals3)


def _proj_call(h_split, W, b2d):
    BM = 2000

    def body(h_ref, w_ref, b_ref, o_ref):
        ha = h_ref[0]
        hb = h_ref[1]
        w = w_ref[...]
        dn = (((1,), (1,)), ((), ()))
        acc = lax.dot_general(ha, w[:, :HALF], dn,
                              preferred_element_type=jnp.float32)
        acc = acc + lax.dot_general(hb, w[:, HALF:], dn,
                                    preferred_element_type=jnp.float32)
        o_ref[...] = jnp.maximum(acc + b_ref[...], 0.0)

    return pl.pallas_call(
        body,
        grid=(N_NODES // BM,),
        in_specs=[
            pl.BlockSpec((NC, BM, HALF), lambda i: (0, i, 0)),
            pl.BlockSpec((N_FEATS, N_FEATS), lambda i: (0, 0)),
            pl.BlockSpec((1, N_FEATS), lambda i: (0, 0)),
        ],
        out_specs=pl.BlockSpec((BM, N_FEATS), lambda i: (i, 0)),
        out_shape=jax.ShapeDtypeStruct((N_NODES, N_FEATS), jnp.float32),
    )(h_split, W, b2d)


def kernel(adj_indices, adj_values, feat_indices, feat_values,
           n_nodes, n_feats, W0, b0, W1, b1, W2, b2):
    del n_nodes, n_feats
    rows = adj_indices[0].astype(jnp.int32)
    cols = adj_indices[1].astype(jnp.int32)
    vals = adj_values.astype(jnp.float32)
    pad = E_PAD - rows.shape[0]
    zi = jnp.zeros((pad,), jnp.int32)
    zf = jnp.zeros((pad,), jnp.float32)
    rows3 = jnp.concatenate([rows, zi]).reshape(NS, NBLK, BLK, K)
    cols3 = jnp.concatenate([cols, zi]).reshape(NS, NBLK, BLK, K)
    vals3 = jnp.concatenate([vals, zf]).reshape(NS, NBLK, BLK, K)

    frows = feat_indices[0].astype(jnp.int32)
    fcols = feat_indices[1].astype(jnp.int32)
    fvals = feat_values.astype(jnp.float32)

    x = _densify_call(frows, fcols, fvals)
    h1 = _spmm_call(x, rows3, cols3, vals3, N_FEATS)
    o1 = _proj_call(h1.reshape(NC, N_NODES, HALF), W0, b0.reshape(1, N_FEATS))
    h2 = _spmm_call(h1, rows3, cols3, vals3, N_NODES)
    o2 = _proj_call(h2.reshape(NC, N_NODES, HALF), W1, b1.reshape(1, N_FEATS))
    h3 = _spmm_call(h2, rows3, cols3, vals3, N_NODES)
    o3 = _proj_call(h3.reshape(NC, N_NODES, HALF), W2, b2.reshape(1, N_FEATS))
    return (o1, o2, o3)


# speedup vs baseline: 4.6491x; 4.6491x over previous
import functools

import jax
import jax.numpy as jnp
from jax import lax
from jax.experimental import pallas as pl
from jax.experimental.pallas import tpu as pltpu
from jax.experimental.pallas import tpu_sc as plsc

N_NODES = 10000
N_FEATS = 256
HALF = 128
NC, NS = 2, 16
ROWS_PER_TILE = N_NODES // NS
K = 64
NCHUNK = 160
BLK = 16
NBLK = NCHUNK // BLK
E_PAD = NS * NCHUNK * K
NNZF = 256000
NNZF_PER_TILE = NNZF // NS
NREP = NS


def _densify_call(frows, fcols, fvals):
    mesh = plsc.VectorSubcoreMesh(core_axis_name="c", subcore_axis_name="s")

    @functools.partial(
        pl.kernel,
        out_type=jax.ShapeDtypeStruct((NC, NREP * N_FEATS, 1, HALF),
                                      jnp.float32),
        mesh=mesh,
        compiler_params=pltpu.CompilerParams(needs_layout_passes=False),
        scratch_types=[
            pltpu.VMEM((NNZF_PER_TILE,), jnp.int32),
            pltpu.VMEM((NNZF_PER_TILE,), jnp.int32),
            pltpu.VMEM((NNZF_PER_TILE,), jnp.float32),
            pltpu.VMEM((N_FEATS, 1, HALF), jnp.float32),
            pltpu.VMEM((HALF,), jnp.int32),
            pltpu.VMEM((HALF,), jnp.int32),
            pltpu.VMEM_SHARED((N_FEATS, 1, HALF), jnp.float32),
        ],
    )
    def k(frows_hbm, fcols_hbm, fvals_hbm, out_hbm,
          rbuf, cbuf, vbuf, acc, idx_lo, idx_hi, shacc):
        c = lax.axis_index("c")
        s = lax.axis_index("s")
        for f in range(HALF // 16):
            idx_lo[pl.ds(f * 16, 16)] = lax.iota(jnp.int32, 16) + f * 16
            idx_hi[pl.ds(f * 16, 16)] = (
                lax.iota(jnp.int32, 16) + (HALF + f * 16))

        def zrow(i, _):
            for f in range(HALF // 16):
                acc[i, 0, pl.ds(f * 16, 16)] = jnp.zeros((16,), jnp.float32)
            return 0
        lax.fori_loop(0, N_FEATS, zrow, 0)

        @pl.when(s == 0)
        def _():
            pltpu.sync_copy(acc, shacc)

        pltpu.sync_copy(frows_hbm.at[pl.ds(s * NNZF_PER_TILE,
                                           NNZF_PER_TILE)], rbuf)
        pltpu.sync_copy(fcols_hbm.at[pl.ds(s * NNZF_PER_TILE,
                                           NNZF_PER_TILE)], cbuf)
        pltpu.sync_copy(fvals_hbm.at[pl.ds(s * NNZF_PER_TILE,
                                           NNZF_PER_TILE)], vbuf)

        def step(i, _):
            off = i * 16
            rv = rbuf[pl.ds(off, 16)]
            cv = cbuf[pl.ds(off, 16)]
            vv = vbuf[pl.ds(off, 16)]
            lc = cv - c * HALF
            m = (lc >= 0) & (lc < HALF)
            lcc = jnp.minimum(jnp.maximum(lc, 0), HALF - 1)
            plsc.addupdate_scatter(acc, [rv, jnp.zeros((16,), jnp.int32),
                                         lcc], vv, mask=m)
            return 0
        lax.fori_loop(0, NNZF_PER_TILE // 16, step, 0)

        plsc.subcore_barrier()
        pltpu.sync_copy(acc.at[pl.ds(0, HALF)], shacc.at[idx_lo], add=True)
        pltpu.sync_copy(acc.at[pl.ds(HALF, HALF)], shacc.at[idx_hi], add=True)
        plsc.subcore_barrier()

        pltpu.sync_copy(shacc, out_hbm.at[c, pl.ds(s * N_FEATS, N_FEATS)])

    return k(frows, fcols, fvals)


def _spmm_call(src, rows3, cols3, vals3, src_rows):
    mesh = plsc.VectorSubcoreMesh(core_axis_name="c", subcore_axis_name="s")
    clip = src_rows < N_NODES

    @functools.partial(
        pl.kernel,
        out_type=jax.ShapeDtypeStruct((NC, N_NODES, 1, HALF), jnp.float32),
        mesh=mesh,
        compiler_params=pltpu.CompilerParams(use_tc_tiling_on_sc=False,
                                             needs_layout_passes=False),
        scratch_types=[
            pltpu.VMEM((BLK, K), jnp.int32),
            pltpu.VMEM((BLK, K), jnp.int32),
            pltpu.VMEM((BLK, K), jnp.float32),
            pltpu.VMEM((K, 1, HALF), jnp.float32),
            pltpu.VMEM((125, 1, HALF), jnp.float32),
            pltpu.VMEM_SHARED((N_NODES, 1, HALF), jnp.float32),
        ],
    )
    def k(src_hbm, rows_hbm, cols_hbm, vals_hbm, out_hbm,
          rbuf, cbuf, vbuf, gbuf, zbuf, acc):
        c = lax.axis_index("c")
        s = lax.axis_index("s")

        def zrow(i, _):
            for f in range(HALF // 16):
                zbuf[i, 0, pl.ds(f * 16, 16)] = jnp.zeros((16,), jnp.float32)
            return 0
        lax.fori_loop(0, 125, zrow, 0)
        for j in range(5):
            pltpu.sync_copy(
                zbuf, acc.at[pl.ds(s * ROWS_PER_TILE + j * 125, 125)])
        plsc.subcore_barrier()

        def block(b, _):
            pltpu.sync_copy(rows_hbm.at[s, b], rbuf)
            pltpu.sync_copy(cols_hbm.at[s, b], cbuf)
            pltpu.sync_copy(vals_hbm.at[s, b], vbuf)

            if clip:
                def fixup(i, _):
                    for j in range(K // 16):
                        cseg = cbuf[i, pl.ds(j * 16, 16)]
                        vseg = vbuf[i, pl.ds(j * 16, 16)]
                        m = cseg < N_FEATS
                        cbuf[i, pl.ds(j * 16, 16)] = (
                            (cseg & (N_FEATS - 1)) + s * N_FEATS)
                        vbuf[i, pl.ds(j * 16, 16)] = jnp.where(
                            m, vseg, jnp.zeros((16,), jnp.float32))
                    return 0
                lax.fori_loop(0, BLK, fixup, 0)

            def chunk(i, _):
                pltpu.sync_copy(src_hbm.at[c].at[cbuf.at[i]], gbuf)

                def group(j, _):
                    vseg = vbuf[i, pl.ds(j * 16, 16)]
                    for l in range(16):
                        sval = vseg[l]
                        e = j * 16 + l
                        for f in range(HALF // 16):
                            gbuf[e, 0, pl.ds(f * 16, 16)] = (
                                gbuf[e, 0, pl.ds(f * 16, 16)] * sval)
                    return 0
                lax.fori_loop(0, K // 16, group, 0)

                pltpu.sync_copy(gbuf, acc.at[rbuf.at[i]], add=True)
                return 0
            lax.fori_loop(0, BLK, chunk, 0)
            return 0
        lax.fori_loop(0, NBLK, block, 0)

        plsc.subcore_barrier()
        pltpu.sync_copy(acc.at[pl.ds(s * ROWS_PER_TILE, ROWS_PER_TILE)],
                        out_hbm.at[c, pl.ds(s * ROWS_PER_TILE, ROWS_PER_TILE)])

    return k(src, rows3, cols3, vals3)


def _proj_call(h_split, W, b2d):
    BM = 2000

    def body(h_ref, w_ref, b_ref, o_ref):
        ha = h_ref[0]
        hb = h_ref[1]
        w = w_ref[...]
        dn = (((1,), (1,)), ((), ()))
        acc = lax.dot_general(ha, w[:, :HALF], dn,
                              preferred_element_type=jnp.float32)
        acc = acc + lax.dot_general(hb, w[:, HALF:], dn,
                                    preferred_element_type=jnp.float32)
        o_ref[...] = jnp.maximum(acc + b_ref[...], 0.0)

    return pl.pallas_call(
        body,
        grid=(N_NODES // BM,),
        in_specs=[
            pl.BlockSpec((NC, BM, HALF), lambda i: (0, i, 0)),
            pl.BlockSpec((N_FEATS, N_FEATS), lambda i: (0, 0)),
            pl.BlockSpec((1, N_FEATS), lambda i: (0, 0)),
        ],
        out_specs=pl.BlockSpec((BM, N_FEATS), lambda i: (i, 0)),
        out_shape=jax.ShapeDtypeStruct((N_NODES, N_FEATS), jnp.float32),
    )(h_split, W, b2d)


def kernel(adj_indices, adj_values, feat_indices, feat_values,
           n_nodes, n_feats, W0, b0, W1, b1, W2, b2):
    del n_nodes, n_feats
    rows = adj_indices[0].astype(jnp.int32)
    cols = adj_indices[1].astype(jnp.int32)
    vals = adj_values.astype(jnp.float32)
    pad = E_PAD - rows.shape[0]
    zi = jnp.zeros((pad,), jnp.int32)
    zf = jnp.zeros((pad,), jnp.float32)
    rows3 = jnp.concatenate([rows, zi]).reshape(NS, NBLK, BLK, K)
    cols3 = jnp.concatenate([cols, zi]).reshape(NS, NBLK, BLK, K)
    vals3 = jnp.concatenate([vals, zf]).reshape(NS, NBLK, BLK, K)

    frows = feat_indices[0].astype(jnp.int32)
    fcols = feat_indices[1].astype(jnp.int32)
    fvals = feat_values.astype(jnp.float32)

    x = _densify_call(frows, fcols, fvals)
    h1 = _spmm_call(x, rows3, cols3, vals3, NREP * N_FEATS)
    o1 = _proj_call(h1.reshape(NC, N_NODES, HALF), W0, b0.reshape(1, N_FEATS))
    h2 = _spmm_call(h1, rows3, cols3, vals3, N_NODES)
    o2 = _proj_call(h2.reshape(NC, N_NODES, HALF), W1, b1.reshape(1, N_FEATS))
    h3 = _spmm_call(h2, rows3, cols3, vals3, N_NODES)
    o3 = _proj_call(h3.reshape(NC, N_NODES, HALF), W2, b2.reshape(1, N_FEATS))
    return (o1, o2, o3)


# speedup vs baseline: 6.1662x; 1.3263x over previous
import functools

import jax
import jax.numpy as jnp
from jax import lax
from jax.experimental import pallas as pl
from jax.experimental.pallas import tpu as pltpu
from jax.experimental.pallas import tpu_sc as plsc

N_NODES = 10000
N_FEATS = 256
HALF = 128
NC, NS = 2, 16
ROWS_PER_TILE = N_NODES // NS
K = 128
NCHUNK = 80
BLK = 8
NBLK = NCHUNK // BLK
E_PAD = NS * NCHUNK * K
NNZF = 256000
NNZF_PER_TILE = NNZF // NS
NREP = NS


def _densify_call(frows, fcols, fvals):
    mesh = plsc.VectorSubcoreMesh(core_axis_name="c", subcore_axis_name="s")

    @functools.partial(
        pl.kernel,
        out_type=jax.ShapeDtypeStruct((NC, NREP * N_FEATS, 1, HALF),
                                      jnp.float32),
        mesh=mesh,
        compiler_params=pltpu.CompilerParams(needs_layout_passes=False),
        scratch_types=[
            pltpu.VMEM((NNZF_PER_TILE,), jnp.int32),
            pltpu.VMEM((NNZF_PER_TILE,), jnp.int32),
            pltpu.VMEM((NNZF_PER_TILE,), jnp.float32),
            pltpu.VMEM((N_FEATS, 1, HALF), jnp.float32),
            pltpu.VMEM((HALF,), jnp.int32),
            pltpu.VMEM((HALF,), jnp.int32),
            pltpu.VMEM_SHARED((N_FEATS, 1, HALF), jnp.float32),
        ],
    )
    def k(frows_hbm, fcols_hbm, fvals_hbm, out_hbm,
          rbuf, cbuf, vbuf, acc, idx_lo, idx_hi, shacc):
        c = lax.axis_index("c")
        s = lax.axis_index("s")
        for f in range(HALF // 16):
            idx_lo[pl.ds(f * 16, 16)] = lax.iota(jnp.int32, 16) + f * 16
            idx_hi[pl.ds(f * 16, 16)] = (
                lax.iota(jnp.int32, 16) + (HALF + f * 16))

        def zrow(i, _):
            for f in range(HALF // 16):
                acc[i, 0, pl.ds(f * 16, 16)] = jnp.zeros((16,), jnp.float32)
            return 0
        lax.fori_loop(0, N_FEATS, zrow, 0)

        @pl.when(s == 0)
        def _():
            pltpu.sync_copy(acc, shacc)

        pltpu.sync_copy(frows_hbm.at[pl.ds(s * NNZF_PER_TILE,
                                           NNZF_PER_TILE)], rbuf)
        pltpu.sync_copy(fcols_hbm.at[pl.ds(s * NNZF_PER_TILE,
                                           NNZF_PER_TILE)], cbuf)
        pltpu.sync_copy(fvals_hbm.at[pl.ds(s * NNZF_PER_TILE,
                                           NNZF_PER_TILE)], vbuf)

        def step(i, _):
            off = i * 16
            rv = rbuf[pl.ds(off, 16)]
            cv = cbuf[pl.ds(off, 16)]
            vv = vbuf[pl.ds(off, 16)]
            lc = cv - c * HALF
            m = (lc >= 0) & (lc < HALF)
            lcc = jnp.minimum(jnp.maximum(lc, 0), HALF - 1)
            plsc.addupdate_scatter(acc, [rv, jnp.zeros((16,), jnp.int32),
                                         lcc], vv, mask=m)
            return 0
        lax.fori_loop(0, NNZF_PER_TILE // 16, step, 0)

        plsc.subcore_barrier()
        pltpu.sync_copy(acc.at[pl.ds(0, HALF)], shacc.at[idx_lo], add=True)
        pltpu.sync_copy(acc.at[pl.ds(HALF, HALF)], shacc.at[idx_hi], add=True)
        plsc.subcore_barrier()

        pltpu.sync_copy(shacc, out_hbm.at[c, pl.ds(s * N_FEATS, N_FEATS)])

    return k(frows, fcols, fvals)


def _spmm_call(src, rows3, cols3, vals3, src_rows):
    mesh = plsc.VectorSubcoreMesh(core_axis_name="c", subcore_axis_name="s")
    clip = src_rows < N_NODES

    @functools.partial(
        pl.kernel,
        out_type=jax.ShapeDtypeStruct((NC, N_NODES, 1, HALF), jnp.float32),
        mesh=mesh,
        compiler_params=pltpu.CompilerParams(use_tc_tiling_on_sc=False,
                                             needs_layout_passes=False),
        scratch_types=[
            pltpu.VMEM((BLK, K), jnp.int32),
            pltpu.VMEM((BLK, K), jnp.int32),
            pltpu.VMEM((BLK, K), jnp.float32),
            pltpu.VMEM((K, 1, HALF), jnp.float32),
            pltpu.VMEM((K, 1, HALF), jnp.float32),
            pltpu.VMEM_SHARED((N_NODES, 1, HALF), jnp.float32),
            pltpu.SemaphoreType.DMA,
            pltpu.SemaphoreType.DMA,
            pltpu.SemaphoreType.DMA,
            pltpu.SemaphoreType.DMA,
        ],
    )
    def k(src_hbm, rows_hbm, cols_hbm, vals_hbm, out_hbm,
          rbuf, cbuf, vbuf, g0, g1, acc, sg0, sg1, ssa, ssb):
        c = lax.axis_index("c")
        s = lax.axis_index("s")

        def zrow(i, _):
            for f in range(HALF // 16):
                g0[i, 0, pl.ds(f * 16, 16)] = jnp.zeros((16,), jnp.float32)
            return 0
        lax.fori_loop(0, K, zrow, 0)
        for j in range(5):
            pltpu.sync_copy(g0.at[pl.ds(0, 125)],
                            acc.at[pl.ds(s * ROWS_PER_TILE + j * 125, 125)])
        plsc.subcore_barrier()

        def scale(gb, i):
            def group(j, _):
                vseg = vbuf[i, pl.ds(j * 16, 16)]
                for l in range(16):
                    sval = vseg[l]
                    e = j * 16 + l
                    for f in range(HALF // 16):
                        gb[e, 0, pl.ds(f * 16, 16)] = (
                            gb[e, 0, pl.ds(f * 16, 16)] * sval)
                return 0
            lax.fori_loop(0, K // 16, group, 0)

        def gather(i, gb, sem):
            return pltpu.async_copy(src_hbm.at[c].at[cbuf.at[i]], gb, sem)

        def scatter(i, gb, sem):
            return pltpu.async_copy(gb, acc.at[rbuf.at[i]], sem, add=True)

        def block(b, _):
            pltpu.sync_copy(rows_hbm.at[s, b], rbuf)
            pltpu.sync_copy(cols_hbm.at[s, b], cbuf)
            pltpu.sync_copy(vals_hbm.at[s, b], vbuf)

            if clip:
                def fixup(i, _):
                    for j in range(K // 16):
                        cseg = cbuf[i, pl.ds(j * 16, 16)]
                        vseg = vbuf[i, pl.ds(j * 16, 16)]
                        m = cseg < N_FEATS
                        cbuf[i, pl.ds(j * 16, 16)] = (
                            (cseg & (N_FEATS - 1)) + s * N_FEATS)
                        vbuf[i, pl.ds(j * 16, 16)] = jnp.where(
                            m, vseg, jnp.zeros((16,), jnp.float32))
                    return 0
                lax.fori_loop(0, BLK, fixup, 0)

            gather(0, g0, sg0)

            def pair(bp, _):
                a = bp * 2
                gather(a + 1, g1, sg1)
                pltpu.make_async_copy(src_hbm.at[c].at[cbuf.at[a]],
                                      g0, sg0).wait()
                scale(g0, a)
                scatter(a, g0, ssa)
                pltpu.make_async_copy(src_hbm.at[c].at[cbuf.at[a + 1]],
                                      g1, sg1).wait()
                scale(g1, a + 1)
                pltpu.make_async_copy(g0, acc.at[rbuf.at[a]], ssa).wait()
                scatter(a + 1, g1, ssb)

                @pl.when(a + 2 < BLK)
                def _():
                    gather(a + 2, g0, sg0)
                pltpu.make_async_copy(g1, acc.at[rbuf.at[a + 1]], ssb).wait()
                return 0
            lax.fori_loop(0, BLK // 2, pair, 0)
            return 0
        lax.fori_loop(0, NBLK, block, 0)

        plsc.subcore_barrier()
        pltpu.sync_copy(acc.at[pl.ds(s * ROWS_PER_TILE, ROWS_PER_TILE)],
                        out_hbm.at[c, pl.ds(s * ROWS_PER_TILE, ROWS_PER_TILE)])

    return k(src, rows3, cols3, vals3)


def _proj_call(h_split, W, b2d):
    BM = 2000

    def body(h_ref, w_ref, b_ref, o_ref):
        ha = h_ref[0]
        hb = h_ref[1]
        w = w_ref[...]
        dn = (((1,), (1,)), ((), ()))
        acc = lax.dot_general(ha, w[:, :HALF], dn,
                              preferred_element_type=jnp.float32)
        acc = acc + lax.dot_general(hb, w[:, HALF:], dn,
                                    preferred_element_type=jnp.float32)
        o_ref[...] = jnp.maximum(acc + b_ref[...], 0.0)

    return pl.pallas_call(
        body,
        grid=(N_NODES // BM,),
        in_specs=[
            pl.BlockSpec((NC, BM, HALF), lambda i: (0, i, 0)),
            pl.BlockSpec((N_FEATS, N_FEATS), lambda i: (0, 0)),
            pl.BlockSpec((1, N_FEATS), lambda i: (0, 0)),
        ],
        out_specs=pl.BlockSpec((BM, N_FEATS), lambda i: (i, 0)),
        out_shape=jax.ShapeDtypeStruct((N_NODES, N_FEATS), jnp.float32),
    )(h_split, W, b2d)


def kernel(adj_indices, adj_values, feat_indices, feat_values,
           n_nodes, n_feats, W0, b0, W1, b1, W2, b2):
    del n_nodes, n_feats
    rows = adj_indices[0].astype(jnp.int32)
    cols = adj_indices[1].astype(jnp.int32)
    vals = adj_values.astype(jnp.float32)
    pad = E_PAD - rows.shape[0]
    zi = jnp.zeros((pad,), jnp.int32)
    zf = jnp.zeros((pad,), jnp.float32)
    rows3 = jnp.concatenate([rows, zi]).reshape(NS, NBLK, BLK, K)
    cols3 = jnp.concatenate([cols, zi]).reshape(NS, NBLK, BLK, K)
    vals3 = jnp.concatenate([vals, zf]).reshape(NS, NBLK, BLK, K)

    frows = feat_indices[0].astype(jnp.int32)
    fcols = feat_indices[1].astype(jnp.int32)
    fvals = feat_values.astype(jnp.float32)

    x = _densify_call(frows, fcols, fvals)
    h1 = _spmm_call(x, rows3, cols3, vals3, NREP * N_FEATS)
    o1 = _proj_call(h1.reshape(NC, N_NODES, HALF), W0, b0.reshape(1, N_FEATS))
    h2 = _spmm_call(h1, rows3, cols3, vals3, N_NODES)
    o2 = _proj_call(h2.reshape(NC, N_NODES, HALF), W1, b1.reshape(1, N_FEATS))
    h3 = _spmm_call(h2, rows3, cols3, vals3, N_NODES)
    o3 = _proj_call(h3.reshape(NC, N_NODES, HALF), W2, b2.reshape(1, N_FEATS))
    return (o1, o2, o3)
